# baseline (speedup 1.0000x reference)
import functools

import jax
import jax.numpy as jnp
from jax import lax
from jax.experimental import pallas as pl
from jax.experimental.pallas import tpu as pltpu
from jax.experimental.pallas import tpu_sc as plsc

_B, _N, _C, _U = 2, 50000, 32, 512
_NPIX = _U * _U
_CHUNK = 512
_NSLAB = _CHUNK // 128
_NW = 32
_TASKS = _B * _NPIX // _CHUNK
_PER_W = _TASKS // _NW
_CH_PER_B = _NPIX // _CHUNK
_WROW = 3 * _NSLAB
_NREC = _WROW + _CHUNK * 3 // 128


def _make_sc_call():
    mesh = plsc.VectorSubcoreMesh(core_axis_name="c", subcore_axis_name="s")

    @functools.partial(
        pl.kernel,
        mesh=mesh,
        out_type=jax.ShapeDtypeStruct((_B * _C, _NPIX), jnp.float32),
        scratch_types=[
            pltpu.VMEM((2, _NREC, 128), jnp.int32),
            pltpu.VMEM((2, 3, _CHUNK, _C), jnp.float32),
            pltpu.VMEM((_C, _CHUNK), jnp.float32),
            pltpu.VMEM((_C // 16 * 16, 16), jnp.int32),
            pltpu.SemaphoreType.DMA,
            pltpu.SemaphoreType.DMA,
            pltpu.SemaphoreType.DMA,
        ],
        compiler_params=pltpu.CompilerParams(
            needs_layout_passes=False, use_tc_tiling_on_sc=False),
    )
    def sc_call(verts_hbm, iw_hbm, out_hbm, iwb, gbuf, acc, cmtab,
                sg0, sg1, so):
        wid = lax.axis_index("s") * 2 + lax.axis_index("c")
        sgs = (sg0, sg1)
        iota = lax.iota(jnp.int32, 16)
        for cg in range(_C // 16):
            for d in range(16):
                cmtab[cg * 16 + d, :] = cg * 16 + ((iota + d) & 15)

        def fire(t, s):
            cid = wid * _PER_W + t
            pltpu.sync_copy(iw_hbm.at[cid], iwb.at[s])
            for kk in range(3):
                for j in range(_NSLAB):
                    pltpu.async_copy(
                        verts_hbm.at[iwb.at[s, kk * _NSLAB + j]],
                        gbuf.at[s, kk, pl.ds(j * 128, 128)], sgs[s])

        def drain_gathers(s):
            for kk in range(3):
                pltpu.make_async_copy(
                    verts_hbm.at[pl.ds(0, _CHUNK)],
                    gbuf.at[s, kk], sgs[s]).wait()

        def drain_out():
            pltpu.make_async_copy(
                out_hbm.at[pl.ds(0, _C), pl.ds(0, _CHUNK)], acc, so).wait()

        def step(t, s):
            nt = t + 1

            @pl.when(nt < _PER_W)
            def _():
                fire(nt, 1 - s)

            drain_gathers(s)

            @pl.when(t > 0)
            def _():
                drain_out()

            @plsc.parallel_loop(0, _CHUNK // 16, unroll=2)
            def g_body(g):
                p0 = g * 16
                rows = p0 + iota
                wr = _WROW + lax.shift_right_logical(g, 3)
                wc = (g & 7) * 16
                ws = []
                for kk in range(3):
                    ws.append(plsc.bitcast(
                        iwb[s, wr + kk * (_CHUNK // 128), pl.ds(wc, 16)],
                        jnp.float32))
                for cg in range(_C // 16):
                    for d in range(16):
                        cols = cmtab[cg * 16 + d, :]
                        v0 = plsc.load_gather(gbuf.at[s, 0], [rows, cols])
                        v1 = plsc.load_gather(gbuf.at[s, 1], [rows, cols])
                        v2 = plsc.load_gather(gbuf.at[s, 2], [rows, cols])
                        tot = v0 * ws[0] + v1 * ws[1] + v2 * ws[2]
                        plsc.store_scatter(acc, [cols, rows], tot)

            cid = wid * _PER_W + t
            b = cid // _CH_PER_B
            base = (cid % _CH_PER_B) * _CHUNK
            pltpu.async_copy(
                acc, out_hbm.at[pl.ds(b * _C, _C), pl.ds(base, _CHUNK)], so)

        fire(0, 0)

        def pair_body(t2, carry):
            step(2 * t2, 0)
            step(2 * t2 + 1, 1)
            return carry

        lax.fori_loop(0, _PER_W // 2, pair_body, 0)
        drain_out()

    return sc_call


_sc_call = _make_sc_call()


@jax.jit
def kernel(verts, pix_to_v, bary_w):
    B, N, C = verts.shape
    U = pix_to_v.shape[0]
    npix = U * U
    verts_ = jnp.concatenate(
        [jnp.zeros((B, 1, C), dtype=verts.dtype), verts], axis=1)
    verts2d = verts_.reshape(B * (N + 1), C)
    idx = jnp.transpose(pix_to_v.reshape(npix, 3), (1, 0)) + 1
    idx = idx[None, :, :] + (
        jnp.arange(B, dtype=jnp.int32) * (N + 1))[:, None, None]
    idx4 = jnp.transpose(
        idx.reshape(B, 3, _CH_PER_B, _NSLAB, 128), (0, 2, 1, 3, 4)
    ).reshape(B * _CH_PER_B, 3 * _NSLAB, 128)
    w2 = jnp.transpose(bary_w.reshape(npix, 3), (1, 0))
    w4 = jax.lax.bitcast_convert_type(
        jnp.transpose(w2.reshape(3, _CH_PER_B, _CHUNK), (1, 0, 2)
                      ).reshape(_CH_PER_B, _NREC - _WROW, 128), jnp.int32)
    w4 = jnp.broadcast_to(
        w4[None], (B, _CH_PER_B, _NREC - _WROW, 128))
    iw = jnp.concatenate(
        [idx4.reshape(B, _CH_PER_B, _WROW, 128), w4], axis=2
    ).reshape(B * _CH_PER_B, _NREC, 128)
    out = _sc_call(verts2d, iw)
    return out.reshape(B, C, U, U)


# speedup vs baseline: 1.2870x; 1.2870x over previous
import functools

import jax
import jax.numpy as jnp
from jax import lax
from jax.experimental import pallas as pl
from jax.experimental.pallas import tpu as pltpu
from jax.experimental.pallas import tpu_sc as plsc

_B, _N, _C, _U = 2, 50000, 32, 512
_NPIX = _U * _U
_CHUNK = 512
_NSLAB = _CHUNK // 128
_NW = 32
_TASKS = _B * _NPIX // _CHUNK
_PER_W = _TASKS // _NW
_CH_PER_B = _NPIX // _CHUNK
_WROW = 3 * _NSLAB
_NREC = _WROW + _CHUNK * 3 // 128


def _make_sc_call():
    mesh = plsc.VectorSubcoreMesh(core_axis_name="c", subcore_axis_name="s")

    @functools.partial(
        pl.kernel,
        mesh=mesh,
        out_type=jax.ShapeDtypeStruct((_B * _C, _NPIX), jnp.float32),
        scratch_types=[
            pltpu.VMEM((2, 2 * _NREC, 128), jnp.int32),
            pltpu.VMEM((2, 3, _CHUNK, _C), jnp.float32),
            pltpu.VMEM((_C, _CHUNK), jnp.float32),
            pltpu.VMEM((_C // 16 * 16, 16), jnp.int32),
            pltpu.SemaphoreType.DMA,
            pltpu.SemaphoreType.DMA,
            pltpu.SemaphoreType.DMA,
        ],
        compiler_params=pltpu.CompilerParams(
            needs_layout_passes=False, use_tc_tiling_on_sc=False),
    )
    def sc_call(verts_hbm, iw_hbm, out_hbm, iwb, gbuf, acc, cmtab,
                sg0, sg1, so):
        wid = lax.axis_index("s") * 2 + lax.axis_index("c")
        sgs = (sg0, sg1)
        iota = lax.iota(jnp.int32, 16)
        for cg in range(_C // 16):
            for d in range(16):
                cmtab[cg * 16 + d, :] = cg * 16 + ((iota + d) & 15)

        def fetch_pair(u, rs):
            pltpu.sync_copy(iw_hbm.at[wid * (_PER_W // 2) + u], iwb.at[rs])

        def fire_gathers(gs, rs, half):
            for kk in range(3):
                for j in range(_NSLAB):
                    pltpu.async_copy(
                        verts_hbm.at[
                            iwb.at[rs, half * _NREC + kk * _NSLAB + j]],
                        gbuf.at[gs, kk, pl.ds(j * 128, 128)], sgs[gs])

        def drain_gathers(s):
            for kk in range(3):
                pltpu.make_async_copy(
                    verts_hbm.at[pl.ds(0, _CHUNK)],
                    gbuf.at[s, kk], sgs[s]).wait()

        def drain_out():
            pltpu.make_async_copy(
                out_hbm.at[pl.ds(0, _C), pl.ds(0, _CHUNK)], acc, so).wait()

        def step(t, s, rs_next, half_next, rs, half, u_next):
            nt = t + 1

            @pl.when(nt < _PER_W)
            def _():
                fire_gathers(1 - s, rs_next, half_next)

            if half == 0:
                @pl.when(u_next < _PER_W // 2)
                def _():
                    fetch_pair(u_next, 1 - rs)

            drain_gathers(s)

            @pl.when(t > 0)
            def _():
                drain_out()

            @plsc.parallel_loop(0, _CHUNK // 16)
            def g_body(g):
                p0 = g * 16
                rows = p0 + iota
                wr = half * _NREC + _WROW + lax.shift_right_logical(g, 3)
                wc = (g & 7) * 16
                ws = []
                for kk in range(3):
                    ws.append(plsc.bitcast(
                        iwb[rs, wr + kk * (_CHUNK // 128), pl.ds(wc, 16)],
                        jnp.float32))
                for cg in range(_C // 16):
                    for d in range(16):
                        cols = cmtab[cg * 16 + d, :]
                        v0 = plsc.load_gather(gbuf.at[s, 0], [rows, cols])
                        v1 = plsc.load_gather(gbuf.at[s, 1], [rows, cols])
                        v2 = plsc.load_gather(gbuf.at[s, 2], [rows, cols])
                        tot = v0 * ws[0] + v1 * ws[1] + v2 * ws[2]
                        plsc.store_scatter(acc, [cols, rows], tot)

            cid = wid * _PER_W + t
            b = cid // _CH_PER_B
            base = (cid % _CH_PER_B) * _CHUNK
            pltpu.async_copy(
                acc, out_hbm.at[pl.ds(b * _C, _C), pl.ds(base, _CHUNK)], so)

        fetch_pair(0, 0)
        fire_gathers(0, 0, 0)

        def quad_body(t4, carry):
            t = 4 * t4
            u = 2 * t4
            step(t, 0, 0, 1, 0, 0, u + 1)
            step(t + 1, 1, 1, 0, 0, 1, u + 1)
            step(t + 2, 0, 1, 1, 1, 0, u + 2)
            step(t + 3, 1, 0, 0, 1, 1, u + 2)
            return carry

        lax.fori_loop(0, _PER_W // 4, quad_body, 0)
        drain_out()

    return sc_call


_sc_call = _make_sc_call()


@jax.jit
def kernel(verts, pix_to_v, bary_w):
    B, N, C = verts.shape
    U = pix_to_v.shape[0]
    npix = U * U
    verts_ = jnp.concatenate(
        [jnp.zeros((B, 1, C), dtype=verts.dtype), verts], axis=1)
    verts2d = verts_.reshape(B * (N + 1), C)
    idx = jnp.transpose(pix_to_v.reshape(npix, 3), (1, 0)) + 1
    idx = idx[None, :, :] + (
        jnp.arange(B, dtype=jnp.int32) * (N + 1))[:, None, None]
    idx4 = jnp.transpose(
        idx.reshape(B, 3, _CH_PER_B, _NSLAB, 128), (0, 2, 1, 3, 4)
    ).reshape(B * _CH_PER_B, 3 * _NSLAB, 128)
    w2 = jnp.transpose(bary_w.reshape(npix, 3), (1, 0))
    w4 = jax.lax.bitcast_convert_type(
        jnp.transpose(w2.reshape(3, _CH_PER_B, _CHUNK), (1, 0, 2)
                      ).reshape(_CH_PER_B, _NREC - _WROW, 128), jnp.int32)
    w4 = jnp.broadcast_to(
        w4[None], (B, _CH_PER_B, _NREC - _WROW, 128))
    iw = jnp.concatenate(
        [idx4.reshape(B, _CH_PER_B, _WROW, 128), w4], axis=2
    ).reshape(B * _CH_PER_B // 2, 2 * _NREC, 128)
    out = _sc_call(verts2d, iw)
    return out.reshape(B, C, U, U)


# speedup vs baseline: 1.3651x; 1.0606x over previous
import functools

import jax
import jax.numpy as jnp
from jax import lax
from jax.experimental import pallas as pl
from jax.experimental.pallas import tpu as pltpu
from jax.experimental.pallas import tpu_sc as plsc

_B, _N, _C, _U = 2, 50000, 32, 512
_NPIX = _U * _U
_CHUNK = 512
_NSLAB = _CHUNK // 128
_NW = 32
_TASKS = _B * _NPIX // _CHUNK
_PER_W = _TASKS // _NW
_CH_PER_B = _NPIX // _CHUNK
_WROW = 3 * _NSLAB
_NREC = _WROW + _CHUNK * 3 // 128


def _make_sc_call():
    mesh = plsc.VectorSubcoreMesh(core_axis_name="c", subcore_axis_name="s")

    @functools.partial(
        pl.kernel,
        mesh=mesh,
        out_type=jax.ShapeDtypeStruct((_B * _C, _NPIX), jnp.float32),
        scratch_types=[
            pltpu.VMEM((2, 2 * _NREC, 128), jnp.int32),
            pltpu.VMEM((2, 3, _CHUNK, _C), jnp.float32),
            pltpu.VMEM((_C, _CHUNK), jnp.float32),
            pltpu.VMEM((16, 16), jnp.int32),
            pltpu.SemaphoreType.DMA,
            pltpu.SemaphoreType.DMA,
            pltpu.SemaphoreType.DMA,
        ],
        compiler_params=pltpu.CompilerParams(
            needs_layout_passes=False, use_tc_tiling_on_sc=False),
    )
    def sc_call(verts_hbm, iw_hbm, out_hbm, iwb, gbuf, acc, cmtab,
                sg0, sg1, so):
        wid = lax.axis_index("s") * 2 + lax.axis_index("c")
        sgs = (sg0, sg1)
        iota = lax.iota(jnp.int32, 16)
        for d in range(16):
            cmtab[d, :] = (iota + d) & 15

        def fetch_pair(u, rs):
            pltpu.sync_copy(iw_hbm.at[wid * (_PER_W // 2) + u], iwb.at[rs])

        def fire_gathers(gs, rs, half):
            for kk in range(3):
                for j in range(_NSLAB):
                    pltpu.async_copy(
                        verts_hbm.at[
                            iwb.at[rs, half * _NREC + kk * _NSLAB + j]],
                        gbuf.at[gs, kk, pl.ds(j * 128, 128)], sgs[gs])

        def drain_gathers(s):
            for kk in range(3):
                pltpu.make_async_copy(
                    verts_hbm.at[pl.ds(0, _CHUNK)],
                    gbuf.at[s, kk], sgs[s]).wait()

        def drain_out():
            pltpu.make_async_copy(
                out_hbm.at[pl.ds(0, _C), pl.ds(0, _CHUNK)], acc, so).wait()

        def step(t, s, rs_next, half_next, rs, half, u_next):
            nt = t + 1

            @pl.when(nt < _PER_W)
            def _():
                fire_gathers(1 - s, rs_next, half_next)

            if half == 0:
                @pl.when(u_next < _PER_W // 2)
                def _():
                    fetch_pair(u_next, 1 - rs)

            drain_gathers(s)

            @pl.when(t > 0)
            def _():
                drain_out()

            @plsc.parallel_loop(0, _CHUNK // 16)
            def g_body(g):
                p0 = g * 16
                rows = p0 + iota
                wr = half * _NREC + _WROW + lax.shift_right_logical(g, 3)
                wc = (g & 7) * 16
                ws = []
                for kk in range(3):
                    ws.append(plsc.bitcast(
                        iwb[rs, wr + kk * (_CHUNK // 128), pl.ds(wc, 16)],
                        jnp.float32))
                for d in range(16):
                    cols0 = cmtab[d, :]
                    for cg in range(_C // 16):
                        cols = cols0 if cg == 0 else cols0 + cg * 16
                        v0 = plsc.load_gather(gbuf.at[s, 0], [rows, cols])
                        v1 = plsc.load_gather(gbuf.at[s, 1], [rows, cols])
                        v2 = plsc.load_gather(gbuf.at[s, 2], [rows, cols])
                        tot = v0 * ws[0] + v1 * ws[1] + v2 * ws[2]
                        plsc.store_scatter(acc, [cols, rows], tot)

            cid = wid * _PER_W + t
            b = cid // _CH_PER_B
            base = (cid % _CH_PER_B) * _CHUNK
            pltpu.async_copy(
                acc, out_hbm.at[pl.ds(b * _C, _C), pl.ds(base, _CHUNK)], so)

        fetch_pair(0, 0)
        fire_gathers(0, 0, 0)

        def quad_body(t4, carry):
            t = 4 * t4
            u = 2 * t4
            step(t, 0, 0, 1, 0, 0, u + 1)
            step(t + 1, 1, 1, 0, 0, 1, u + 1)
            step(t + 2, 0, 1, 1, 1, 0, u + 2)
            step(t + 3, 1, 0, 0, 1, 1, u + 2)
            return carry

        lax.fori_loop(0, _PER_W // 4, quad_body, 0)
        drain_out()

    return sc_call


_sc_call = _make_sc_call()


@jax.jit
def kernel(verts, pix_to_v, bary_w):
    B, N, C = verts.shape
    U = pix_to_v.shape[0]
    npix = U * U
    verts_ = jnp.concatenate(
        [jnp.zeros((B, 1, C), dtype=verts.dtype), verts], axis=1)
    verts2d = verts_.reshape(B * (N + 1), C)
    idx = jnp.transpose(pix_to_v.reshape(npix, 3), (1, 0)) + 1
    idx = idx[None, :, :] + (
        jnp.arange(B, dtype=jnp.int32) * (N + 1))[:, None, None]
    idx4 = jnp.transpose(
        idx.reshape(B, 3, _CH_PER_B, _NSLAB, 128), (0, 2, 1, 3, 4)
    ).reshape(B * _CH_PER_B, 3 * _NSLAB, 128)
    w2 = jnp.transpose(bary_w.reshape(npix, 3), (1, 0))
    w4 = jax.lax.bitcast_convert_type(
        jnp.transpose(w2.reshape(3, _CH_PER_B, _CHUNK), (1, 0, 2)
                      ).reshape(_CH_PER_B, _NREC - _WROW, 128), jnp.int32)
    w4 = jnp.broadcast_to(
        w4[None], (B, _CH_PER_B, _NREC - _WROW, 128))
    iw = jnp.concatenate(
        [idx4.reshape(B, _CH_PER_B, _WROW, 128), w4], axis=2
    ).reshape(B * _CH_PER_B // 2, 2 * _NREC, 128)
    out = _sc_call(verts2d, iw)
    return out.reshape(B, C, U, U)
